# baseline (speedup 1.0000x reference)
import jax
import jax.numpy as jnp
from jax import lax
from jax.experimental import pallas as pl
from jax.experimental.pallas import tpu as pltpu
from jax.experimental.pallas import tpu_sc as plsc

_D = 128
_B = 16384
_EPS = 1e-5

_NC = 2
_NS = 16
_NW = _NC * _NS
_B_PER_W = _B // _NW
_L = 16
_NCG = _D // _L
_NCHUNK = 4
_CH = _B_PER_W // _NCHUNK


def _xsum16(v):
    for sh in (8, 4, 2, 1):
        perm = lax.iota(jnp.int32, _L) ^ sh
        v = v + v.at[perm].get(mode="promise_in_bounds")
    return v


def _rsqrt16(v):
    s = v[0]
    bits = lax.bitcast_convert_type(s, jnp.int32)
    y0 = lax.bitcast_convert_type(jnp.int32(0x5F3759DF) - (bits >> 1),
                                  jnp.float32)
    y = jnp.full((_L,), y0, jnp.float32)
    half = 0.5 * v
    for _ in range(6):
        y = y * (1.5 - half * y * y)
    return y


def _dense_rows(emb_v, wt_v, b_v, g_v, beta_v):
    out = []
    for r in range(2):
        acc = [b_v[pl.ds(_L * c, _L)] for c in range(_NCG)]

        def kgroup(g, acc):
            ev = emb_v[r, pl.ds(g * _L, _L)]
            for l in range(_L):
                ef = jnp.full((_L,), ev[l], jnp.float32)
                k = g * _L + l
                for c in range(_NCG):
                    acc[c] = acc[c] + ef * wt_v[k, pl.ds(_L * c, _L)]
            return acc

        acc = lax.fori_loop(0, _NCG, kgroup, acc)
        total = acc[0]
        for c in range(1, _NCG):
            total = total + acc[c]
        mu = _xsum16(total) * (1.0 / _D)
        d = [acc[c] - mu for c in range(_NCG)]
        sq = d[0] * d[0]
        for c in range(1, _NCG):
            sq = sq + d[c] * d[c]
        var = _xsum16(sq) * (1.0 / _D)
        scale = _rsqrt16(var + _EPS)
        out.append([d[c] * scale * g_v[pl.ds(_L * c, _L)]
                    + beta_v[pl.ds(_L * c, _L)] for c in range(_NCG)])
    return out


def _body(x_hbm, emb_hbm, wt_hbm, b_hbm, g_hbm, beta_hbm, out_hbm,
          emb_v, wt_v, b_v, g_v, beta_v, t2_v, t2_sh, idx_v, rows_v, sem):
    cid = lax.axis_index("c")
    sid = lax.axis_index("s")
    wid = sid * _NC + cid
    base = wid * _B_PER_W

    @pl.when(sid == 0)
    def _compute_table():
        pltpu.sync_copy(emb_hbm, emb_v)
        pltpu.sync_copy(wt_hbm, wt_v)
        pltpu.sync_copy(b_hbm, b_v)
        pltpu.sync_copy(g_hbm, g_v)
        pltpu.sync_copy(beta_hbm, beta_v)
        rows = _dense_rows(emb_v, wt_v, b_v, g_v, beta_v)
        for r in range(2):
            for c in range(_NCG):
                t2_v[r, pl.ds(_L * c, _L)] = rows[r][c]
        pltpu.sync_copy(t2_v, t2_sh)

    pltpu.sync_copy(x_hbm.at[pl.ds(base, _B_PER_W)], idx_v)
    plsc.subcore_barrier()
    pltpu.sync_copy(t2_sh, t2_v)

    r0 = [t2_v[0, pl.ds(_L * c, _L)] for c in range(_NCG)]
    dl = [t2_v[1, pl.ds(_L * c, _L)] - r0[c] for c in range(_NCG)]

    copies = []
    for ch in range(_NCHUNK):
        def jbody(j, carry, ch=ch):
            xv = idx_v[pl.ds(ch * _CH + j * _L, _L)].astype(jnp.float32)
            for l in range(_L):
                xf = jnp.full((_L,), xv[l], jnp.float32)
                i = ch * _CH + j * _L + l
                for c in range(_NCG):
                    rows_v[i, pl.ds(_L * c, _L)] = r0[c] + xf * dl[c]
            return carry

        lax.fori_loop(0, _CH // _L, jbody, 0)
        copies.append(pltpu.async_copy(
            rows_v.at[pl.ds(ch * _CH, _CH)],
            out_hbm.at[pl.ds(base + ch * _CH, _CH)], sem))
    for c in copies:
        c.wait()


def _lookup_call():
    return pl.kernel(
        _body,
        out_type=jax.ShapeDtypeStruct((_B, _D), jnp.float32),
        mesh=plsc.VectorSubcoreMesh(core_axis_name="c", subcore_axis_name="s",
                                    num_cores=_NC, num_subcores=_NS),
        scratch_types=[
            pltpu.VMEM((2, _D), jnp.float32),
            pltpu.VMEM((_D, _D), jnp.float32),
            pltpu.VMEM((_D,), jnp.float32),
            pltpu.VMEM((_D,), jnp.float32),
            pltpu.VMEM((_D,), jnp.float32),
            pltpu.VMEM((2, _D), jnp.float32),
            pltpu.VMEM_SHARED((2, _D), jnp.float32),
            pltpu.VMEM((_B_PER_W,), jnp.int32),
            pltpu.VMEM((_B_PER_W, _D), jnp.float32),
            pltpu.SemaphoreType.DMA,
        ],
    )


def kernel(x, emb_table, W, b, gamma, beta):
    idx = x.astype(jnp.int32)
    wt = W.T
    return _lookup_call()(idx, emb_table, wt, b, gamma, beta)


# speedup vs baseline: 1.0918x; 1.0918x over previous
import functools

import jax
import jax.numpy as jnp
from jax import lax
from jax.experimental import pallas as pl
from jax.experimental.pallas import tpu as pltpu
from jax.experimental.pallas import tpu_sc as plsc

_D = 128
_B = 16384
_EPS = 1e-5

_NC = 2
_NS = 16
_NW = _NC * _NS
_B_PER_W = _B // _NW


def _dense_body(emb_ref, w_ref, b_ref, g_ref, beta_ref, out_ref):
    emb = emb_ref[...]
    w = w_ref[...]
    h = lax.dot_general(emb, w, (((1,), (1,)), ((), ())),
                        preferred_element_type=jnp.float32)
    h = h + b_ref[...]
    mu = jnp.mean(h, axis=-1, keepdims=True)
    d = h - mu
    var = jnp.mean(d * d, axis=-1, keepdims=True)
    out_ref[...] = d * lax.rsqrt(var + _EPS) * g_ref[...] + beta_ref[...]


_dense_call = pl.pallas_call(
    _dense_body,
    out_shape=jax.ShapeDtypeStruct((8, _D), jnp.float32),
)


_L = 16
_NCG = _D // _L
_NCHUNK = 8
_CH = _B_PER_W // _NCHUNK


def _gather_body(table_hbm, idx_hbm, out_hbm, table_v, idx_v, rows_v, sem):
    wid = lax.axis_index("s") * _NC + lax.axis_index("c")
    base = wid * _B_PER_W
    pltpu.sync_copy(table_hbm.at[pl.ds(0, 8)], table_v)
    pltpu.sync_copy(idx_hbm.at[pl.ds(base, _B_PER_W)], idx_v)
    r0 = [table_v[0, pl.ds(_L * c, _L)] for c in range(_NCG)]
    dl = [table_v[1, pl.ds(_L * c, _L)] - r0[c] for c in range(_NCG)]

    copies = []
    for ch in range(_NCHUNK):
        def body(j, carry, ch=ch):
            xv = idx_v[pl.ds(ch * _CH + j * _L, _L)].astype(jnp.float32)
            for l in range(_L):
                xf = jnp.full((_L,), xv[l], jnp.float32)
                i = ch * _CH + j * _L + l
                for c in range(_NCG):
                    rows_v[i, pl.ds(_L * c, _L)] = r0[c] + xf * dl[c]
            return carry

        lax.fori_loop(0, _CH // _L, body, 0)
        copies.append(pltpu.async_copy(
            rows_v.at[pl.ds(ch * _CH, _CH)],
            out_hbm.at[pl.ds(base + ch * _CH, _CH)], sem))
    for c in copies:
        c.wait()


@functools.cache
def _gather_call():
    return pl.kernel(
        _gather_body,
        out_type=jax.ShapeDtypeStruct((_B, _D), jnp.float32),
        mesh=plsc.VectorSubcoreMesh(core_axis_name="c", subcore_axis_name="s",
                                    num_cores=_NC, num_subcores=_NS),
        scratch_types=[
            pltpu.VMEM((8, _D), jnp.float32),
            pltpu.VMEM((_B_PER_W,), jnp.int32),
            pltpu.VMEM((_B_PER_W, _D), jnp.float32),
            pltpu.SemaphoreType.DMA,
        ],
    )


def kernel(x, emb_table, W, b, gamma, beta):
    idx = x.astype(jnp.int32)
    emb_pad = jnp.pad(emb_table, ((0, 8 - emb_table.shape[0]), (0, 0)))
    table = _dense_call(emb_pad, W, b.reshape(1, _D),
                        gamma.reshape(1, _D), beta.reshape(1, _D))
    return _gather_call()(table, idx)


# speedup vs baseline: 1.2316x; 1.1280x over previous
import functools

import jax
import jax.numpy as jnp
from jax import lax
from jax.experimental import pallas as pl
from jax.experimental.pallas import tpu as pltpu
from jax.experimental.pallas import tpu_sc as plsc

_D = 128
_B = 16384
_EPS = 1e-5

_NC = 2
_NS = 16
_NW = _NC * _NS
_B_PER_W = _B // _NW


def _dense_body(emb_ref, w_ref, b_ref, g_ref, beta_ref, out_ref):
    emb = emb_ref[...]
    w = w_ref[...]
    h = lax.dot_general(emb, w, (((1,), (1,)), ((), ())),
                        preferred_element_type=jnp.float32)
    h = h + b_ref[...]
    mu = jnp.mean(h, axis=-1, keepdims=True)
    d = h - mu
    var = jnp.mean(d * d, axis=-1, keepdims=True)
    out_ref[...] = d * lax.rsqrt(var + _EPS) * g_ref[...] + beta_ref[...]


_dense_call = pl.pallas_call(
    _dense_body,
    out_shape=jax.ShapeDtypeStruct((2, _D), jnp.float32),
)


_L = 16
_NCG = _D // _L
_NCHUNK = 2
_CH = _B_PER_W // _NCHUNK


def _gather_body(table_hbm, idx_hbm, out_hbm, table_v, idx_v, rows_v, sem):
    wid = lax.axis_index("s") * _NC + lax.axis_index("c")
    base = wid * _B_PER_W
    pltpu.sync_copy(table_hbm, table_v)
    pltpu.sync_copy(idx_hbm.at[pl.ds(base, _B_PER_W)], idx_v)
    r0 = [table_v[0, pl.ds(_L * c, _L)] for c in range(_NCG)]
    dl = [table_v[1, pl.ds(_L * c, _L)] - r0[c] for c in range(_NCG)]

    copies = []
    for ch in range(_NCHUNK):
        def body(j, carry, ch=ch):
            xv = idx_v[pl.ds(ch * _CH + j * _L, _L)].astype(jnp.float32)
            for l in range(_L):
                xf = jnp.full((_L,), xv[l], jnp.float32)
                i = ch * _CH + j * _L + l
                for c in range(_NCG):
                    rows_v[i, pl.ds(_L * c, _L)] = r0[c] + xf * dl[c]
            return carry

        lax.fori_loop(0, _CH // _L, body, 0)
        copies.append(pltpu.async_copy(
            rows_v.at[pl.ds(ch * _CH, _CH)],
            out_hbm.at[pl.ds(base + ch * _CH, _CH)], sem))
    for c in copies:
        c.wait()


@functools.cache
def _gather_call():
    return pl.kernel(
        _gather_body,
        out_type=jax.ShapeDtypeStruct((_B, _D), jnp.float32),
        mesh=plsc.VectorSubcoreMesh(core_axis_name="c", subcore_axis_name="s",
                                    num_cores=_NC, num_subcores=_NS),
        scratch_types=[
            pltpu.VMEM((2, _D), jnp.float32),
            pltpu.VMEM((_B_PER_W,), jnp.int32),
            pltpu.VMEM((_B_PER_W, _D), jnp.float32),
            pltpu.SemaphoreType.DMA,
        ],
    )


def kernel(x, emb_table, W, b, gamma, beta):
    idx = x.astype(jnp.int32)
    table = _dense_call(emb_table, W, b.reshape(1, _D),
                        gamma.reshape(1, _D), beta.reshape(1, _D))
    return _gather_call()(table, idx)


# speedup vs baseline: 1.2634x; 1.0258x over previous
import functools

import jax
import jax.numpy as jnp
from jax import lax
from jax.experimental import pallas as pl
from jax.experimental.pallas import tpu as pltpu
from jax.experimental.pallas import tpu_sc as plsc

_D = 128
_B = 16384
_EPS = 1e-5

_NC = 2
_NS = 16
_NW = _NC * _NS
_B_PER_W = _B // _NW


def _dense_body(emb_ref, w_ref, b_ref, g_ref, beta_ref, out_ref):
    emb = emb_ref[...]
    w = w_ref[...]
    h = lax.dot_general(emb, w, (((1,), (1,)), ((), ())),
                        preferred_element_type=jnp.float32)
    h = h + b_ref[...]
    mu = jnp.mean(h, axis=-1, keepdims=True)
    d = h - mu
    var = jnp.mean(d * d, axis=-1, keepdims=True)
    out_ref[...] = d * lax.rsqrt(var + _EPS) * g_ref[...] + beta_ref[...]


_dense_call = pl.pallas_call(
    _dense_body,
    out_shape=jax.ShapeDtypeStruct((2, _D), jnp.float32),
)


_L = 16
_NCG = _D // _L
_NCHUNK = 2
_CH = _B_PER_W // _NCHUNK


def _gather_body(table_hbm, idx_hbm, out_hbm, table_v, idx_v, rows_v, sem):
    wid = lax.axis_index("s") * _NC + lax.axis_index("c")
    base = wid * _B_PER_W
    pltpu.sync_copy(table_hbm, table_v)
    pltpu.sync_copy(idx_hbm.at[pl.ds(base, _B_PER_W)], idx_v)
    r0 = [table_v[0, pl.ds(_L * c, _L)] for c in range(_NCG)]
    dl = [table_v[1, pl.ds(_L * c, _L)] - r0[c] for c in range(_NCG)]

    copies = []
    for ch in range(_NCHUNK):
        def body(j, carry, ch=ch):
            xv = idx_v[pl.ds(ch * _CH + j * _L, _L)].astype(jnp.float32)
            for l in range(_L):
                xf = jnp.full((_L,), xv[l], jnp.float32)
                i = ch * _CH + j * _L + l
                for c in range(_NCG):
                    rows_v[i, pl.ds(_L * c, _L)] = r0[c] + xf * dl[c]
            return carry

        lax.fori_loop(0, _CH // _L, body, 0)
        copies.append(pltpu.async_copy(
            rows_v.at[pl.ds(ch * _CH, _CH)],
            out_hbm.at[pl.ds(base + ch * _CH, _CH)], sem))
    for c in copies:
        c.wait()


@functools.cache
def _gather_call():
    return pl.kernel(
        _gather_body,
        out_type=jax.ShapeDtypeStruct((_B, _D), jnp.float32),
        mesh=plsc.VectorSubcoreMesh(core_axis_name="c", subcore_axis_name="s",
                                    num_cores=_NC, num_subcores=_NS),
        scratch_types=[
            pltpu.VMEM((2, _D), jnp.float32),
            pltpu.VMEM((_B_PER_W,), jnp.int32),
            pltpu.VMEM((_B_PER_W, _D), jnp.float32),
            pltpu.SemaphoreType.DMA,
        ],
    )


_BT = 1024


def _select_body(x_ref, t_ref, o_ref):
    xf = x_ref[0]
    t0 = t_ref[0:1, :]
    d = t_ref[1:2, :] - t0
    o_ref[...] = t0 + xf * d


def _select_call(nrows):
    grid = nrows // _BT
    return pl.pallas_call(
        _select_body,
        grid=(grid,),
        in_specs=[
            pl.BlockSpec((1, _BT, 1), lambda i: (i, 0, 0)),
            pl.BlockSpec((2, _D), lambda i: (0, 0)),
        ],
        out_specs=pl.BlockSpec((_BT, _D), lambda i: (i, 0)),
        out_shape=jax.ShapeDtypeStruct((nrows, _D), jnp.float32),
    )


def kernel(x, emb_table, W, b, gamma, beta):
    idx = x.astype(jnp.int32)
    table = _dense_call(emb_table, W, b.reshape(1, _D),
                        gamma.reshape(1, _D), beta.reshape(1, _D))
    xf = idx.astype(jnp.float32).reshape(_B // _BT, _BT, 1)
    return _select_call(_B)(xf, table)
